# baseline (speedup 1.0000x reference)
import functools

import jax
import jax.numpy as jnp
from jax import lax
from jax.experimental import pallas as pl
from jax.experimental.pallas import tpu as pltpu
from jax.experimental.pallas import tpu_sc as plsc

B = 16384
D = 64
DP = 128
V = 100000
TPB = 1024
CHUNK = 128


def _tp_body(u_ref, i_ref, ou_ref, oi_ref):
  z = jnp.zeros((TPB, D), jnp.float32)
  ou_ref[...] = jnp.concatenate([u_ref[...].T, z], axis=1)
  oi_ref[...] = jnp.concatenate([i_ref[...].T, z], axis=1)


def _transpose_tables(uet, iet):
  grid = (pl.cdiv(V, TPB),)
  in_spec = pl.BlockSpec((D, TPB), lambda c: (0, c))
  out_spec = pl.BlockSpec((TPB, DP), lambda c: (c, 0))
  return pl.pallas_call(
      _tp_body,
      grid=grid,
      in_specs=[in_spec, in_spec],
      out_specs=[out_spec, out_spec],
      out_shape=[jax.ShapeDtypeStruct((V, DP), jnp.float32)] * 2,
      compiler_params=pltpu.CompilerParams(
          dimension_semantics=("arbitrary",),
      ),
  )(uet, iet)


def _make_sc_gather():
  info = plsc.get_sparse_core_info()
  nc, ns = info.num_cores, info.num_subcores
  nw = nc * ns
  b_per_w = B // nw
  n_chunks = b_per_w // CHUNK

  mesh = plsc.VectorSubcoreMesh(core_axis_name="c", subcore_axis_name="s")

  @functools.partial(
      pl.kernel,
      mesh=mesh,
      compiler_params=pltpu.CompilerParams(use_tc_tiling_on_sc=False),
      out_type=[
          jax.ShapeDtypeStruct((B, 2 * DP), jnp.float32),
          jax.ShapeDtypeStruct((B,), jnp.float32),
          jax.ShapeDtypeStruct((B,), jnp.float32),
      ],
      scratch_types=[
          pltpu.VMEM((n_chunks, CHUNK), jnp.int32),
          pltpu.VMEM((n_chunks, CHUNK), jnp.int32),
          pltpu.VMEM((b_per_w // 2, DP), jnp.float32),
          pltpu.VMEM((b_per_w // 2, DP), jnp.float32),
          pltpu.VMEM((b_per_w,), jnp.float32),
          pltpu.VMEM((b_per_w,), jnp.float32),
          pltpu.SemaphoreType.DMA,
      ],
  )
  def gather_kernel(uidx_hbm, iidx_hbm, uemb_hbm, iemb_hbm, ubias_hbm,
                    ibias_hbm, out_x, out_ub, out_ib,
                    uidx_v, iidx_v, ue_v, ie_v, ub_v, ib_v, sem):
    wid = lax.axis_index("s") * nc + lax.axis_index("c")
    base = wid * b_per_w
    row0 = wid * n_chunks

    pltpu.sync_copy(uidx_hbm.at[pl.ds(row0, n_chunks)], uidx_v)
    pltpu.sync_copy(iidx_hbm.at[pl.ds(row0, n_chunks)], iidx_v)

    bias_copies = []
    for j in range(n_chunks):
      sl = pl.ds(j * CHUNK, CHUNK)
      bias_copies.append(
          pltpu.async_copy(ubias_hbm.at[uidx_v.at[j]], ub_v.at[sl], sem))
      bias_copies.append(
          pltpu.async_copy(ibias_hbm.at[iidx_v.at[j]], ib_v.at[sl], sem))

    half = n_chunks // 2
    for p in range(2):
      copies = []
      for q in range(half):
        j = p * half + q
        sl = pl.ds(q * CHUNK, CHUNK)
        copies.append(
            pltpu.async_copy(uemb_hbm.at[uidx_v.at[j]], ue_v.at[sl], sem))
        copies.append(
            pltpu.async_copy(iemb_hbm.at[iidx_v.at[j]], ie_v.at[sl], sem))
      for c in copies:
        c.wait()
      out_sl = pl.ds(base + p * half * CHUNK, half * CHUNK)
      pltpu.sync_copy(ue_v, out_x.at[out_sl, pl.ds(0, DP)])
      pltpu.sync_copy(ie_v, out_x.at[out_sl, pl.ds(DP, DP)])

    for c in bias_copies:
      c.wait()
    out_sl2 = pl.ds(base, b_per_w)
    pltpu.sync_copy(ub_v, out_ub.at[out_sl2])
    pltpu.sync_copy(ib_v, out_ib.at[out_sl2])

  return gather_kernel


_sc_gather = _make_sc_gather()


def _mlp_body(xin, ub, ib, w0, b0, w1, b1, w2, b2, w3, b3, w4, b4,
              wo, bo, out):
  f32 = jnp.float32
  x = jnp.dot(xin[...], w0[...], preferred_element_type=f32)
  x = jnp.maximum(x + b0[...], 0.0)
  x = jnp.maximum(jnp.dot(x, w1[...], preferred_element_type=f32) + b1[...], 0.0)
  x = jnp.maximum(jnp.dot(x, w2[...], preferred_element_type=f32) + b2[...], 0.0)
  x = jnp.maximum(jnp.dot(x, w3[...], preferred_element_type=f32) + b3[...], 0.0)
  x = jnp.maximum(jnp.dot(x, w4[...], preferred_element_type=f32) + b4[...], 0.0)
  o = jnp.dot(x, wo[...], preferred_element_type=f32)
  out[...] = o + bo[...] + ub[...] + ib[...]


def _mlp(x, ub, ib, w0, b0, w1, b1, w2, b2, w3, b3, w4, b4, wo, bo,
         blk=8192):
  grid = (B // blk,)

  def data_spec(n):
    return pl.BlockSpec((blk, n), lambda i: (i, 0))

  def w_spec(m, n):
    return pl.BlockSpec((m, n), lambda i: (0, 0))

  return pl.pallas_call(
      _mlp_body,
      grid=grid,
      in_specs=[
          data_spec(2 * DP), data_spec(1), data_spec(1),
          w_spec(2 * DP, 128), w_spec(1, 128),
          w_spec(128, 256), w_spec(1, 256),
          w_spec(256, 128), w_spec(1, 128),
          w_spec(128, 64), w_spec(1, 64),
          w_spec(64, 32), w_spec(1, 32),
          w_spec(32, 1), w_spec(1, 1),
      ],
      out_specs=data_spec(1),
      out_shape=jax.ShapeDtypeStruct((B, 1), jnp.float32),
      compiler_params=pltpu.CompilerParams(
          dimension_semantics=("arbitrary",),
      ),
  )(x, ub, ib, w0, b0, w1, b1, w2, b2, w3, b3, w4, b4, wo, bo)


def kernel(user_idx, item_idx, user_embed, item_embed, user_bias, item_bias,
           W0, b0, W1, b1, W2, b2, W3, b3, W4, b4, Wo, bo):
  uidx = user_idx.astype(jnp.int32).reshape(B // CHUNK, CHUNK)
  iidx = item_idx.astype(jnp.int32).reshape(B // CHUNK, CHUNK)

  uemb_p, iemb_p = _transpose_tables(user_embed.T, item_embed.T)

  x, ub, ib = _sc_gather(uidx, iidx, uemb_p, iemb_p,
                         user_bias.reshape(-1), item_bias.reshape(-1))
  ub = ub.reshape(B, 1)
  ib = ib.reshape(B, 1)

  w0z = jnp.zeros((2 * DP, 128), dtype=W0.dtype)
  w0z = w0z.at[0:D].set(W0[0:D])
  w0z = w0z.at[DP:DP + D].set(W0[D:2 * D])

  out = _mlp(x, ub, ib, w0z, b0.reshape(1, -1),
             W1, b1.reshape(1, -1), W2, b2.reshape(1, -1),
             W3, b3.reshape(1, -1), W4, b4.reshape(1, -1),
             Wo, bo.reshape(1, 1))
  return out


# speedup vs baseline: 1.2732x; 1.2732x over previous
import functools

import jax
import jax.numpy as jnp
from jax import lax
from jax.experimental import pallas as pl
from jax.experimental.pallas import tpu as pltpu
from jax.experimental.pallas import tpu_sc as plsc

B = 16384
D = 64
DP = 128
V = 100000
TPB = 4096
CHUNK = 128


def _tp_body(u_ref, i_ref, ou_ref, oi_ref):
  z = jnp.zeros((TPB, D), jnp.float32)
  ou_ref[...] = jnp.concatenate([u_ref[...].T, z], axis=1)
  oi_ref[...] = jnp.concatenate([i_ref[...].T, z], axis=1)


def _transpose_tables(uet, iet):
  grid = (pl.cdiv(V, TPB),)
  in_spec = pl.BlockSpec((D, TPB), lambda c: (0, c))
  out_spec = pl.BlockSpec((TPB, DP), lambda c: (c, 0))
  return pl.pallas_call(
      _tp_body,
      grid=grid,
      in_specs=[in_spec, in_spec],
      out_specs=[out_spec, out_spec],
      out_shape=[jax.ShapeDtypeStruct((V, DP), jnp.float32)] * 2,
      compiler_params=pltpu.CompilerParams(
          dimension_semantics=("arbitrary",),
      ),
  )(uet, iet)


def _make_sc_gather():
  info = plsc.get_sparse_core_info()
  nc, ns = info.num_cores, info.num_subcores
  nw = nc * ns
  b_per_w = B // nw
  n_chunks = b_per_w // CHUNK

  mesh = plsc.VectorSubcoreMesh(core_axis_name="c", subcore_axis_name="s")

  @functools.partial(
      pl.kernel,
      mesh=mesh,
      compiler_params=pltpu.CompilerParams(use_tc_tiling_on_sc=False),
      out_type=[
          jax.ShapeDtypeStruct((B, 2 * DP), jnp.float32),
          jax.ShapeDtypeStruct((B,), jnp.float32),
          jax.ShapeDtypeStruct((B,), jnp.float32),
      ],
      scratch_types=[
          pltpu.VMEM((n_chunks, CHUNK), jnp.int32),
          pltpu.VMEM((n_chunks, CHUNK), jnp.int32),
          pltpu.VMEM((b_per_w // 2, DP), jnp.float32),
          pltpu.VMEM((b_per_w // 2, DP), jnp.float32),
          pltpu.VMEM((b_per_w,), jnp.float32),
          pltpu.VMEM((b_per_w,), jnp.float32),
          pltpu.SemaphoreType.DMA,
      ],
  )
  def gather_kernel(uidx_hbm, iidx_hbm, uemb_hbm, iemb_hbm, ubias_hbm,
                    ibias_hbm, out_x, out_ub, out_ib,
                    uidx_v, iidx_v, ue_v, ie_v, ub_v, ib_v, sem):
    wid = lax.axis_index("s") * nc + lax.axis_index("c")
    base = wid * b_per_w
    row0 = wid * n_chunks

    pltpu.sync_copy(uidx_hbm.at[pl.ds(row0, n_chunks)], uidx_v)
    pltpu.sync_copy(iidx_hbm.at[pl.ds(row0, n_chunks)], iidx_v)

    bias_copies = []
    for j in range(n_chunks):
      sl = pl.ds(j * CHUNK, CHUNK)
      bias_copies.append(
          pltpu.async_copy(ubias_hbm.at[uidx_v.at[j]], ub_v.at[sl], sem))
      bias_copies.append(
          pltpu.async_copy(ibias_hbm.at[iidx_v.at[j]], ib_v.at[sl], sem))

    half = n_chunks // 2
    for p in range(2):
      copies = []
      for q in range(half):
        j = p * half + q
        sl = pl.ds(q * CHUNK, CHUNK)
        copies.append(
            pltpu.async_copy(uemb_hbm.at[uidx_v.at[j]], ue_v.at[sl], sem))
        copies.append(
            pltpu.async_copy(iemb_hbm.at[iidx_v.at[j]], ie_v.at[sl], sem))
      for c in copies:
        c.wait()
      out_sl = pl.ds(base + p * half * CHUNK, half * CHUNK)
      pltpu.sync_copy(ue_v, out_x.at[out_sl, pl.ds(0, DP)])
      pltpu.sync_copy(ie_v, out_x.at[out_sl, pl.ds(DP, DP)])

    for c in bias_copies:
      c.wait()
    out_sl2 = pl.ds(base, b_per_w)
    pltpu.sync_copy(ub_v, out_ub.at[out_sl2])
    pltpu.sync_copy(ib_v, out_ib.at[out_sl2])

  return gather_kernel


_sc_gather = _make_sc_gather()


def _mlp_body(xin, ub, ib, w0, b0, w1, b1, w2, b2, w3, b3, w4, b4,
              wo, bo, out):
  f32 = jnp.float32
  x = jnp.dot(xin[...], w0[...], preferred_element_type=f32)
  x = jnp.maximum(x + b0[...], 0.0)
  x = jnp.maximum(jnp.dot(x, w1[...], preferred_element_type=f32) + b1[...], 0.0)
  x = jnp.maximum(jnp.dot(x, w2[...], preferred_element_type=f32) + b2[...], 0.0)
  x = jnp.maximum(jnp.dot(x, w3[...], preferred_element_type=f32) + b3[...], 0.0)
  x = jnp.maximum(jnp.dot(x, w4[...], preferred_element_type=f32) + b4[...], 0.0)
  o = jnp.dot(x, wo[...], preferred_element_type=f32)
  out[...] = o + bo[...] + ub[...] + ib[...]


def _mlp(x, ub, ib, w0, b0, w1, b1, w2, b2, w3, b3, w4, b4, wo, bo,
         blk=8192):
  grid = (B // blk,)

  def data_spec(n):
    return pl.BlockSpec((blk, n), lambda i: (i, 0))

  def w_spec(m, n):
    return pl.BlockSpec((m, n), lambda i: (0, 0))

  return pl.pallas_call(
      _mlp_body,
      grid=grid,
      in_specs=[
          data_spec(2 * DP), data_spec(1), data_spec(1),
          w_spec(2 * DP, 128), w_spec(1, 128),
          w_spec(128, 256), w_spec(1, 256),
          w_spec(256, 128), w_spec(1, 128),
          w_spec(128, 64), w_spec(1, 64),
          w_spec(64, 32), w_spec(1, 32),
          w_spec(32, 1), w_spec(1, 1),
      ],
      out_specs=data_spec(1),
      out_shape=jax.ShapeDtypeStruct((B, 1), jnp.float32),
      compiler_params=pltpu.CompilerParams(
          dimension_semantics=("arbitrary",),
      ),
  )(x, ub, ib, w0, b0, w1, b1, w2, b2, w3, b3, w4, b4, wo, bo)


def kernel(user_idx, item_idx, user_embed, item_embed, user_bias, item_bias,
           W0, b0, W1, b1, W2, b2, W3, b3, W4, b4, Wo, bo):
  uidx = user_idx.astype(jnp.int32).reshape(B // CHUNK, CHUNK)
  iidx = item_idx.astype(jnp.int32).reshape(B // CHUNK, CHUNK)

  uemb_p, iemb_p = _transpose_tables(user_embed.T, item_embed.T)

  x, ub, ib = _sc_gather(uidx, iidx, uemb_p, iemb_p,
                         user_bias.reshape(-1), item_bias.reshape(-1))
  ub = ub.reshape(B, 1)
  ib = ib.reshape(B, 1)

  w0z = jnp.zeros((2 * DP, 128), dtype=W0.dtype)
  w0z = w0z.at[0:D].set(W0[0:D])
  w0z = w0z.at[DP:DP + D].set(W0[D:2 * D])

  out = _mlp(x, ub, ib, w0z, b0.reshape(1, -1),
             W1, b1.reshape(1, -1), W2, b2.reshape(1, -1),
             W3, b3.reshape(1, -1), W4, b4.reshape(1, -1),
             Wo, bo.reshape(1, 1))
  return out


# speedup vs baseline: 1.3076x; 1.0270x over previous
import functools

import jax
import jax.numpy as jnp
from jax import lax
from jax.experimental import pallas as pl
from jax.experimental.pallas import tpu as pltpu
from jax.experimental.pallas import tpu_sc as plsc

B = 16384
D = 64
DP = 128
V = 100000
TPB = 8192
CHUNK = 128


def _tp_body(u_ref, i_ref, ou_ref, oi_ref):
  z = jnp.zeros((TPB, D), jnp.float32)
  ou_ref[...] = jnp.concatenate([u_ref[...].T, z], axis=1)
  oi_ref[...] = jnp.concatenate([i_ref[...].T, z], axis=1)


def _transpose_tables(uet, iet):
  grid = (pl.cdiv(V, TPB),)
  in_spec = pl.BlockSpec((D, TPB), lambda c: (0, c))
  out_spec = pl.BlockSpec((TPB, DP), lambda c: (c, 0))
  return pl.pallas_call(
      _tp_body,
      grid=grid,
      in_specs=[in_spec, in_spec],
      out_specs=[out_spec, out_spec],
      out_shape=[jax.ShapeDtypeStruct((V, DP), jnp.float32)] * 2,
      compiler_params=pltpu.CompilerParams(
          dimension_semantics=("arbitrary",),
      ),
  )(uet, iet)


def _make_sc_gather():
  info = plsc.get_sparse_core_info()
  nc, ns = info.num_cores, info.num_subcores
  nw = nc * ns
  b_per_w = B // nw
  n_chunks = b_per_w // CHUNK

  mesh = plsc.VectorSubcoreMesh(core_axis_name="c", subcore_axis_name="s")

  @functools.partial(
      pl.kernel,
      mesh=mesh,
      compiler_params=pltpu.CompilerParams(use_tc_tiling_on_sc=False),
      out_type=[
          jax.ShapeDtypeStruct((B, 2 * DP), jnp.float32),
          jax.ShapeDtypeStruct((B,), jnp.float32),
          jax.ShapeDtypeStruct((B,), jnp.float32),
      ],
      scratch_types=[
          pltpu.VMEM((n_chunks, CHUNK), jnp.int32),
          pltpu.VMEM((n_chunks, CHUNK), jnp.int32),
          pltpu.VMEM((b_per_w // 2, DP), jnp.float32),
          pltpu.VMEM((b_per_w // 2, DP), jnp.float32),
          pltpu.VMEM((b_per_w,), jnp.float32),
          pltpu.VMEM((b_per_w,), jnp.float32),
          pltpu.SemaphoreType.DMA,
      ],
  )
  def gather_kernel(uidx_hbm, iidx_hbm, uemb_hbm, iemb_hbm, ubias_hbm,
                    ibias_hbm, out_x, out_ub, out_ib,
                    uidx_v, iidx_v, ue_v, ie_v, ub_v, ib_v, sem):
    wid = lax.axis_index("s") * nc + lax.axis_index("c")
    base = wid * b_per_w
    row0 = wid * n_chunks

    pltpu.sync_copy(uidx_hbm.at[pl.ds(row0, n_chunks)], uidx_v)
    pltpu.sync_copy(iidx_hbm.at[pl.ds(row0, n_chunks)], iidx_v)

    bias_copies = []
    for j in range(n_chunks):
      sl = pl.ds(j * CHUNK, CHUNK)
      bias_copies.append(
          pltpu.async_copy(ubias_hbm.at[uidx_v.at[j]], ub_v.at[sl], sem))
      bias_copies.append(
          pltpu.async_copy(ibias_hbm.at[iidx_v.at[j]], ib_v.at[sl], sem))

    half = n_chunks // 2
    for p in range(2):
      copies = []
      for q in range(half):
        j = p * half + q
        sl = pl.ds(q * CHUNK, CHUNK)
        copies.append(
            pltpu.async_copy(uemb_hbm.at[uidx_v.at[j]], ue_v.at[sl], sem))
        copies.append(
            pltpu.async_copy(iemb_hbm.at[iidx_v.at[j]], ie_v.at[sl], sem))
      for c in copies:
        c.wait()
      out_sl = pl.ds(base + p * half * CHUNK, half * CHUNK)
      pltpu.sync_copy(ue_v, out_x.at[out_sl, pl.ds(0, DP)])
      pltpu.sync_copy(ie_v, out_x.at[out_sl, pl.ds(DP, DP)])

    for c in bias_copies:
      c.wait()
    out_sl2 = pl.ds(base, b_per_w)
    pltpu.sync_copy(ub_v, out_ub.at[out_sl2])
    pltpu.sync_copy(ib_v, out_ib.at[out_sl2])

  return gather_kernel


_sc_gather = _make_sc_gather()


def _mlp_body(xin, ub, ib, w0, b0, w1, b1, w2, b2, w3, b3, w4, b4,
              wo, bo, out):
  f32 = jnp.float32
  x = jnp.dot(xin[...], w0[...], preferred_element_type=f32)
  x = jnp.maximum(x + b0[...], 0.0)
  x = jnp.maximum(jnp.dot(x, w1[...], preferred_element_type=f32) + b1[...], 0.0)
  x = jnp.maximum(jnp.dot(x, w2[...], preferred_element_type=f32) + b2[...], 0.0)
  x = jnp.maximum(jnp.dot(x, w3[...], preferred_element_type=f32) + b3[...], 0.0)
  x = jnp.maximum(jnp.dot(x, w4[...], preferred_element_type=f32) + b4[...], 0.0)
  o = jnp.dot(x, wo[...], preferred_element_type=f32)
  out[...] = o + bo[...] + ub[...] + ib[...]


def _mlp(x, ub, ib, w0, b0, w1, b1, w2, b2, w3, b3, w4, b4, wo, bo,
         blk=8192):
  grid = (B // blk,)

  def data_spec(n):
    return pl.BlockSpec((blk, n), lambda i: (i, 0))

  def w_spec(m, n):
    return pl.BlockSpec((m, n), lambda i: (0, 0))

  return pl.pallas_call(
      _mlp_body,
      grid=grid,
      in_specs=[
          data_spec(2 * DP), data_spec(1), data_spec(1),
          w_spec(2 * DP, 128), w_spec(1, 128),
          w_spec(128, 256), w_spec(1, 256),
          w_spec(256, 128), w_spec(1, 128),
          w_spec(128, 64), w_spec(1, 64),
          w_spec(64, 32), w_spec(1, 32),
          w_spec(32, 1), w_spec(1, 1),
      ],
      out_specs=data_spec(1),
      out_shape=jax.ShapeDtypeStruct((B, 1), jnp.float32),
      compiler_params=pltpu.CompilerParams(
          dimension_semantics=("arbitrary",),
      ),
  )(x, ub, ib, w0, b0, w1, b1, w2, b2, w3, b3, w4, b4, wo, bo)


def kernel(user_idx, item_idx, user_embed, item_embed, user_bias, item_bias,
           W0, b0, W1, b1, W2, b2, W3, b3, W4, b4, Wo, bo):
  uidx = user_idx.astype(jnp.int32).reshape(B // CHUNK, CHUNK)
  iidx = item_idx.astype(jnp.int32).reshape(B // CHUNK, CHUNK)

  uemb_p, iemb_p = _transpose_tables(user_embed.T, item_embed.T)

  x, ub, ib = _sc_gather(uidx, iidx, uemb_p, iemb_p,
                         user_bias.reshape(-1), item_bias.reshape(-1))
  ub = ub.reshape(B, 1)
  ib = ib.reshape(B, 1)

  w0z = jnp.zeros((2 * DP, 128), dtype=W0.dtype)
  w0z = w0z.at[0:D].set(W0[0:D])
  w0z = w0z.at[DP:DP + D].set(W0[D:2 * D])

  out = _mlp(x, ub, ib, w0z, b0.reshape(1, -1),
             W1, b1.reshape(1, -1), W2, b2.reshape(1, -1),
             W3, b3.reshape(1, -1), W4, b4.reshape(1, -1),
             Wo, bo.reshape(1, 1))
  return out


# speedup vs baseline: 1.3133x; 1.0043x over previous
import functools

import jax
import jax.numpy as jnp
from jax import lax
from jax.experimental import pallas as pl
from jax.experimental.pallas import tpu as pltpu
from jax.experimental.pallas import tpu_sc as plsc

B = 16384
D = 64
DP = 128
V = 100000
TPB = 16384
CHUNK = 128


def _tp_body(u_ref, i_ref, ou_ref, oi_ref):
  z = jnp.zeros((TPB, D), jnp.float32)
  ou_ref[...] = jnp.concatenate([u_ref[...].T, z], axis=1)
  oi_ref[...] = jnp.concatenate([i_ref[...].T, z], axis=1)


def _transpose_tables(uet, iet):
  grid = (pl.cdiv(V, TPB),)
  in_spec = pl.BlockSpec((D, TPB), lambda c: (0, c))
  out_spec = pl.BlockSpec((TPB, DP), lambda c: (c, 0))
  return pl.pallas_call(
      _tp_body,
      grid=grid,
      in_specs=[in_spec, in_spec],
      out_specs=[out_spec, out_spec],
      out_shape=[jax.ShapeDtypeStruct((V, DP), jnp.float32)] * 2,
      compiler_params=pltpu.CompilerParams(
          dimension_semantics=("arbitrary",),
      ),
  )(uet, iet)


def _make_sc_gather():
  info = plsc.get_sparse_core_info()
  nc, ns = info.num_cores, info.num_subcores
  nw = nc * ns
  b_per_w = B // nw
  n_chunks = b_per_w // CHUNK

  mesh = plsc.VectorSubcoreMesh(core_axis_name="c", subcore_axis_name="s")

  @functools.partial(
      pl.kernel,
      mesh=mesh,
      compiler_params=pltpu.CompilerParams(use_tc_tiling_on_sc=False),
      out_type=[
          jax.ShapeDtypeStruct((B, 2 * DP), jnp.float32),
          jax.ShapeDtypeStruct((B,), jnp.float32),
          jax.ShapeDtypeStruct((B,), jnp.float32),
      ],
      scratch_types=[
          pltpu.VMEM((n_chunks, CHUNK), jnp.int32),
          pltpu.VMEM((n_chunks, CHUNK), jnp.int32),
          pltpu.VMEM((b_per_w // 2, DP), jnp.float32),
          pltpu.VMEM((b_per_w // 2, DP), jnp.float32),
          pltpu.VMEM((b_per_w,), jnp.float32),
          pltpu.VMEM((b_per_w,), jnp.float32),
          pltpu.SemaphoreType.DMA,
      ],
  )
  def gather_kernel(uidx_hbm, iidx_hbm, uemb_hbm, iemb_hbm, ubias_hbm,
                    ibias_hbm, out_x, out_ub, out_ib,
                    uidx_v, iidx_v, ue_v, ie_v, ub_v, ib_v, sem):
    wid = lax.axis_index("s") * nc + lax.axis_index("c")
    base = wid * b_per_w
    row0 = wid * n_chunks

    pltpu.sync_copy(uidx_hbm.at[pl.ds(row0, n_chunks)], uidx_v)
    pltpu.sync_copy(iidx_hbm.at[pl.ds(row0, n_chunks)], iidx_v)

    bias_copies = []
    for j in range(n_chunks):
      sl = pl.ds(j * CHUNK, CHUNK)
      bias_copies.append(
          pltpu.async_copy(ubias_hbm.at[uidx_v.at[j]], ub_v.at[sl], sem))
      bias_copies.append(
          pltpu.async_copy(ibias_hbm.at[iidx_v.at[j]], ib_v.at[sl], sem))

    half = n_chunks // 2
    for p in range(2):
      copies = []
      for q in range(half):
        j = p * half + q
        sl = pl.ds(q * CHUNK, CHUNK)
        copies.append(
            pltpu.async_copy(uemb_hbm.at[uidx_v.at[j]], ue_v.at[sl], sem))
        copies.append(
            pltpu.async_copy(iemb_hbm.at[iidx_v.at[j]], ie_v.at[sl], sem))
      for c in copies:
        c.wait()
      out_sl = pl.ds(base + p * half * CHUNK, half * CHUNK)
      pltpu.sync_copy(ue_v, out_x.at[out_sl, pl.ds(0, DP)])
      pltpu.sync_copy(ie_v, out_x.at[out_sl, pl.ds(DP, DP)])

    for c in bias_copies:
      c.wait()
    out_sl2 = pl.ds(base, b_per_w)
    pltpu.sync_copy(ub_v, out_ub.at[out_sl2])
    pltpu.sync_copy(ib_v, out_ib.at[out_sl2])

  return gather_kernel


_sc_gather = _make_sc_gather()


def _mlp_body(xin, ub, ib, w0, b0, w1, b1, w2, b2, w3, b3, w4, b4,
              wo, bo, out):
  f32 = jnp.float32
  x = jnp.dot(xin[...], w0[...], preferred_element_type=f32)
  x = jnp.maximum(x + b0[...], 0.0)
  x = jnp.maximum(jnp.dot(x, w1[...], preferred_element_type=f32) + b1[...], 0.0)
  x = jnp.maximum(jnp.dot(x, w2[...], preferred_element_type=f32) + b2[...], 0.0)
  x = jnp.maximum(jnp.dot(x, w3[...], preferred_element_type=f32) + b3[...], 0.0)
  x = jnp.maximum(jnp.dot(x, w4[...], preferred_element_type=f32) + b4[...], 0.0)
  o = jnp.dot(x, wo[...], preferred_element_type=f32)
  out[...] = o + bo[...] + ub[...] + ib[...]


def _mlp(x, ub, ib, w0, b0, w1, b1, w2, b2, w3, b3, w4, b4, wo, bo,
         blk=8192):
  grid = (B // blk,)

  def data_spec(n):
    return pl.BlockSpec((blk, n), lambda i: (i, 0))

  def w_spec(m, n):
    return pl.BlockSpec((m, n), lambda i: (0, 0))

  return pl.pallas_call(
      _mlp_body,
      grid=grid,
      in_specs=[
          data_spec(2 * DP), data_spec(1), data_spec(1),
          w_spec(2 * DP, 128), w_spec(1, 128),
          w_spec(128, 256), w_spec(1, 256),
          w_spec(256, 128), w_spec(1, 128),
          w_spec(128, 64), w_spec(1, 64),
          w_spec(64, 32), w_spec(1, 32),
          w_spec(32, 1), w_spec(1, 1),
      ],
      out_specs=data_spec(1),
      out_shape=jax.ShapeDtypeStruct((B, 1), jnp.float32),
      compiler_params=pltpu.CompilerParams(
          dimension_semantics=("arbitrary",),
      ),
  )(x, ub, ib, w0, b0, w1, b1, w2, b2, w3, b3, w4, b4, wo, bo)


def kernel(user_idx, item_idx, user_embed, item_embed, user_bias, item_bias,
           W0, b0, W1, b1, W2, b2, W3, b3, W4, b4, Wo, bo):
  uidx = user_idx.astype(jnp.int32).reshape(B // CHUNK, CHUNK)
  iidx = item_idx.astype(jnp.int32).reshape(B // CHUNK, CHUNK)

  uemb_p, iemb_p = _transpose_tables(user_embed.T, item_embed.T)

  x, ub, ib = _sc_gather(uidx, iidx, uemb_p, iemb_p,
                         user_bias.reshape(-1), item_bias.reshape(-1))
  ub = ub.reshape(B, 1)
  ib = ib.reshape(B, 1)

  w0z = jnp.zeros((2 * DP, 128), dtype=W0.dtype)
  w0z = w0z.at[0:D].set(W0[0:D])
  w0z = w0z.at[DP:DP + D].set(W0[D:2 * D])

  out = _mlp(x, ub, ib, w0z, b0.reshape(1, -1),
             W1, b1.reshape(1, -1), W2, b2.reshape(1, -1),
             W3, b3.reshape(1, -1), W4, b4.reshape(1, -1),
             Wo, bo.reshape(1, 1))
  return out


# speedup vs baseline: 1.3297x; 1.0125x over previous
import functools

import jax
import jax.numpy as jnp
from jax import lax
from jax.experimental import pallas as pl
from jax.experimental.pallas import tpu as pltpu
from jax.experimental.pallas import tpu_sc as plsc

B = 16384
D = 64
DP = 128
V = 100000
TPB = 16384
CHUNK = 128


def _tp_body(u_ref, i_ref, ou_ref, oi_ref):
  z = jnp.zeros((TPB, D), jnp.float32)
  ou_ref[...] = jnp.concatenate([u_ref[...].T, z], axis=1)
  oi_ref[...] = jnp.concatenate([i_ref[...].T, z], axis=1)


def _transpose_tables(uet, iet):
  grid = (pl.cdiv(V, TPB),)
  in_spec = pl.BlockSpec((D, TPB), lambda c: (0, c))
  out_spec = pl.BlockSpec((TPB, DP), lambda c: (c, 0))
  return pl.pallas_call(
      _tp_body,
      grid=grid,
      in_specs=[in_spec, in_spec],
      out_specs=[out_spec, out_spec],
      out_shape=[jax.ShapeDtypeStruct((V, DP), jnp.float32)] * 2,
      compiler_params=pltpu.CompilerParams(
          dimension_semantics=("arbitrary",),
      ),
  )(uet, iet)


def _make_sc_gather():
  info = plsc.get_sparse_core_info()
  nc, ns = info.num_cores, info.num_subcores
  nw = nc * ns
  b_per_w = B // nw
  n_chunks = b_per_w // CHUNK

  mesh = plsc.VectorSubcoreMesh(core_axis_name="c", subcore_axis_name="s")

  @functools.partial(
      pl.kernel,
      mesh=mesh,
      compiler_params=pltpu.CompilerParams(use_tc_tiling_on_sc=False),
      out_type=[
          jax.ShapeDtypeStruct((B, 2 * DP), jnp.float32),
          jax.ShapeDtypeStruct((B,), jnp.float32),
          jax.ShapeDtypeStruct((B,), jnp.float32),
      ],
      scratch_types=[
          pltpu.VMEM((n_chunks, CHUNK), jnp.int32),
          pltpu.VMEM((n_chunks, CHUNK), jnp.int32),
          pltpu.VMEM((b_per_w // 2, DP), jnp.float32),
          pltpu.VMEM((b_per_w // 2, DP), jnp.float32),
          pltpu.VMEM((b_per_w,), jnp.float32),
          pltpu.VMEM((b_per_w,), jnp.float32),
          pltpu.SemaphoreType.DMA,
      ],
  )
  def gather_kernel(uidx_hbm, iidx_hbm, uemb_hbm, iemb_hbm, ubias_hbm,
                    ibias_hbm, out_x, out_ub, out_ib,
                    uidx_v, iidx_v, ue_v, ie_v, ub_v, ib_v, sem):
    wid = lax.axis_index("s") * nc + lax.axis_index("c")
    base = wid * b_per_w
    row0 = wid * n_chunks

    pltpu.sync_copy(uidx_hbm.at[pl.ds(row0, n_chunks)], uidx_v)
    pltpu.sync_copy(iidx_hbm.at[pl.ds(row0, n_chunks)], iidx_v)

    bias_copies = []
    for j in range(n_chunks):
      sl = pl.ds(j * CHUNK, CHUNK)
      bias_copies.append(
          pltpu.async_copy(ubias_hbm.at[uidx_v.at[j]], ub_v.at[sl], sem))
      bias_copies.append(
          pltpu.async_copy(ibias_hbm.at[iidx_v.at[j]], ib_v.at[sl], sem))

    half = n_chunks // 2
    for p in range(2):
      copies = []
      for q in range(half):
        j = p * half + q
        sl = pl.ds(q * CHUNK, CHUNK)
        copies.append(
            pltpu.async_copy(uemb_hbm.at[uidx_v.at[j]], ue_v.at[sl], sem))
        copies.append(
            pltpu.async_copy(iemb_hbm.at[iidx_v.at[j]], ie_v.at[sl], sem))
      for c in copies:
        c.wait()
      out_sl = pl.ds(base + p * half * CHUNK, half * CHUNK)
      pltpu.sync_copy(ue_v, out_x.at[out_sl, pl.ds(0, DP)])
      pltpu.sync_copy(ie_v, out_x.at[out_sl, pl.ds(DP, DP)])

    for c in bias_copies:
      c.wait()
    out_sl2 = pl.ds(base, b_per_w)
    pltpu.sync_copy(ub_v, out_ub.at[out_sl2])
    pltpu.sync_copy(ib_v, out_ib.at[out_sl2])

  return gather_kernel


_sc_gather = _make_sc_gather()


def _mlp_body(xin, ub, ib, w0, b0, w1, b1, w2, b2, w3, b3, w4, b4,
              wo, bo, out):
  f32 = jnp.float32
  x = jnp.dot(xin[...], w0[...], preferred_element_type=f32)
  x = jnp.maximum(x + b0[...], 0.0)
  x = jnp.maximum(jnp.dot(x, w1[...], preferred_element_type=f32) + b1[...], 0.0)
  x = jnp.maximum(jnp.dot(x, w2[...], preferred_element_type=f32) + b2[...], 0.0)
  x = jnp.maximum(jnp.dot(x, w3[...], preferred_element_type=f32) + b3[...], 0.0)
  x = jnp.maximum(jnp.dot(x, w4[...], preferred_element_type=f32) + b4[...], 0.0)
  o = jnp.dot(x, wo[...], preferred_element_type=f32)
  out[...] = o + bo[...] + ub[...] + ib[...]


def _mlp(x, ub, ib, w0, b0, w1, b1, w2, b2, w3, b3, w4, b4, wo, bo,
         blk=4096):
  grid = (B // blk,)

  def data_spec(n):
    return pl.BlockSpec((blk, n), lambda i: (i, 0))

  def w_spec(m, n):
    return pl.BlockSpec((m, n), lambda i: (0, 0))

  return pl.pallas_call(
      _mlp_body,
      grid=grid,
      in_specs=[
          data_spec(2 * DP), data_spec(1), data_spec(1),
          w_spec(2 * DP, 128), w_spec(1, 128),
          w_spec(128, 256), w_spec(1, 256),
          w_spec(256, 128), w_spec(1, 128),
          w_spec(128, 64), w_spec(1, 64),
          w_spec(64, 32), w_spec(1, 32),
          w_spec(32, 1), w_spec(1, 1),
      ],
      out_specs=data_spec(1),
      out_shape=jax.ShapeDtypeStruct((B, 1), jnp.float32),
      compiler_params=pltpu.CompilerParams(
          dimension_semantics=("arbitrary",),
      ),
  )(x, ub, ib, w0, b0, w1, b1, w2, b2, w3, b3, w4, b4, wo, bo)


def kernel(user_idx, item_idx, user_embed, item_embed, user_bias, item_bias,
           W0, b0, W1, b1, W2, b2, W3, b3, W4, b4, Wo, bo):
  uidx = user_idx.astype(jnp.int32).reshape(B // CHUNK, CHUNK)
  iidx = item_idx.astype(jnp.int32).reshape(B // CHUNK, CHUNK)

  uemb_p, iemb_p = _transpose_tables(user_embed.T, item_embed.T)

  x, ub, ib = _sc_gather(uidx, iidx, uemb_p, iemb_p,
                         user_bias.reshape(-1), item_bias.reshape(-1))
  ub = ub.reshape(B, 1)
  ib = ib.reshape(B, 1)

  w0z = jnp.zeros((2 * DP, 128), dtype=W0.dtype)
  w0z = w0z.at[0:D].set(W0[0:D])
  w0z = w0z.at[DP:DP + D].set(W0[D:2 * D])

  out = _mlp(x, ub, ib, w0z, b0.reshape(1, -1),
             W1, b1.reshape(1, -1), W2, b2.reshape(1, -1),
             W3, b3.reshape(1, -1), W4, b4.reshape(1, -1),
             Wo, bo.reshape(1, 1))
  return out
